# baseline (speedup 1.0000x reference)
import functools

import jax
import jax.numpy as jnp
from jax import lax
from jax.experimental import pallas as pl
from jax.experimental.pallas import tpu as pltpu
from jax.experimental.pallas import tpu_sc as plsc

B = 16384
DIM = 64
NV = 1000000
NC, NS = 2, 16
NW = NC * NS
NGT = (NV + 127) // 128
GPW = 245
RING = 6
POOL = 16
ROW_BYTES = DIM * 4


def _sc_id_gather(vertex_ids, id_emb_w):
    tbl_t = id_emb_w.T
    tail_t = id_emb_w[NV - 128:].T
    mesh = plsc.VectorSubcoreMesh(
        core_axis_name="c", subcore_axis_name="s", num_cores=NC, num_subcores=NS)

    @functools.partial(
        pl.kernel,
        mesh=mesh,
        compiler_params=pltpu.CompilerParams(
            needs_layout_passes=False, disable_bounds_checks=True),
        out_type=jax.ShapeDtypeStruct((B * DIM,), jnp.float32),
        scratch_types=[
            pltpu.VMEM((B,), jnp.int32),
            pltpu.VMEM((B + 16,), jnp.int32),
            pltpu.VMEM((B + 16,), jnp.int32),
            pltpu.VMEM((256,), jnp.int32),
            pltpu.VMEM((256,), jnp.int32),
            pltpu.VMEM((256,), jnp.int32),
            pltpu.VMEM((272,), jnp.int32),
            pltpu.VMEM((RING, DIM, 128), jnp.float32),
            pltpu.VMEM((RING, POOL, DIM), jnp.float32),
            pltpu.SemaphoreType.DMA,
            pltpu.SemaphoreType.DMA,
            pltpu.SemaphoreType.DMA,
            pltpu.SemaphoreType.DMA,
            pltpu.SemaphoreType.DMA,
            pltpu.SemaphoreType.DMA,
            pltpu.SemaphoreType.DMA,
            pltpu.SemaphoreType.DMA,
            pltpu.SemaphoreType.DMA,
            pltpu.SemaphoreType.DMA,
            pltpu.SemaphoreType.DMA,
            pltpu.SemaphoreType.DMA,
        ],
    )
    def k(ids_hbm, tbl_hbm, tail_hbm, out_hbm, ids_v, spack_v, ssort_v,
          cnt_v, end_v, wr_v, nzg_v, blk_v, pool_v,
          bsem0, bsem1, bsem2, bsem3, bsem4, bsem5,
          rsem0, rsem1, rsem2, rsem3, rsem4, rsem5):
        bsems = (bsem0, bsem1, bsem2, bsem3, bsem4, bsem5)
        rsems = (rsem0, rsem1, rsem2, rsem3, rsem4, rsem5)
        wid = lax.axis_index("s") * NC + lax.axis_index("c")
        g_lo = wid * GPW
        g_hi = jnp.minimum(g_lo + GPW, NGT)
        iota = lax.iota(jnp.int32, 16)
        ones = jnp.ones((16,), jnp.int32)
        zeros = jnp.zeros((16,), jnp.int32)

        pltpu.sync_copy(ids_hbm, ids_v)
        for kk in range(16):
            cnt_v[pl.ds(kk * 16, 16)] = zeros

        def filt(i, off):
            v = ids_v[pl.ds(i * 16, 16)]
            grp = lax.shift_right_logical(v, 7)
            m = (grp >= g_lo) & (grp < g_hi)
            c = plsc.all_reduce_population_count(m)
            lane = (v & 127) + jnp.where(grp == NGT - 1, 64, 0)
            gl = jnp.where(m, grp - g_lo, 0)
            packed = (gl << 21) | ((i * 16 + iota) << 7) | lane
            plsc.store_compressed(spack_v.at[pl.ds(off, 16)], packed, mask=m)
            plsc.addupdate_scatter(cnt_v, [gl], ones, mask=m)
            return off + c[0]

        n = lax.fori_loop(0, B // 16, filt, 0)

        carry = 0
        moff = 0
        for kk in range(16):
            v = cnt_v[pl.ds(kk * 16, 16)]
            cs = plsc.cumsum(v) + carry
            end_v[pl.ds(kk * 16, 16)] = cs
            wr_v[pl.ds(kk * 16, 16)] = cs - v
            m = v > 0
            c = plsc.all_reduce_population_count(m)
            plsc.store_compressed(nzg_v.at[pl.ds(moff, 16)], kk * 16 + iota, mask=m)
            moff = moff + c[0]
            carry = cs[15]
        m_w = moff

        def srt(i, _):
            e = spack_v[pl.ds(i * 16, 16)]
            valid = iota < (n - i * 16)
            gl = jnp.where(valid, lax.shift_right_logical(e, 21), 0)
            base = plsc.load_gather(wr_v, [gl])
            rank, _last = plsc.scan_count(gl, mask=valid)
            dst = base + rank - 1
            plsc.store_scatter(ssort_v, [dst], e, mask=valid)
            plsc.addupdate_scatter(wr_v, [gl], ones, mask=valid)
            return 0

        lax.fori_loop(0, (n + 15) >> 4, srt, 0)

        def fetch(slot, list_idx):
            gl = nzg_v[pl.ds(jnp.minimum(list_idx, m_w - 1), 16)][0]
            gabs = g_lo + gl

            @pl.when(gabs < NGT - 1)
            def _():
                pltpu.async_copy(
                    tbl_hbm.at[:, pl.ds(gabs * 128, 128)], blk_v.at[slot],
                    bsems[slot])

            @pl.when(gabs == NGT - 1)
            def _():
                pltpu.async_copy(tail_hbm, blk_v.at[slot], bsems[slot])

        @pl.when(m_w > 0)
        def _():
            for r in range(RING):
                fetch(r, r)

            def triad(i, outst):
                new_outst = []
                for r in range(RING):
                    o_r = outst[r]

                    def drain1(_j, _c):
                        pltpu.make_async_copy(
                            out_hbm.at[pl.ds(0, DIM)],
                            pool_v.at[r, 0, pl.ds(0, DIM)], rsems[r]).wait()
                        return 0

                    lax.fori_loop(0, o_r, drain1, 0)
                    pltpu.make_async_copy(
                        tbl_hbm.at[:, pl.ds(0, 128)], blk_v.at[r],
                        bsems[r]).wait()

                    idx = jnp.minimum(i * RING + r, m_w - 1)
                    gl = nzg_v[pl.ds(idx, 16)][0]
                    e_prev = end_v[pl.ds(jnp.maximum(gl - 1, 0), 16)][0]
                    seg_lo = jnp.where(gl == 0, 0, e_prev)
                    seg_hi = end_v[pl.ds(gl, 16)][0]

                    def row(s, rows):
                        b = lax.rem(rows, POOL)

                        @pl.when((b == 0) & (rows > 0))
                        def _():
                            lax.fori_loop(0, POOL, drain1, 0)

                        e = ssort_v[pl.ds(s, 16)][0]
                        j = e & 127
                        p = lax.shift_right_logical(e, 7) & (B - 1)
                        j16 = jnp.full((16,), 0, jnp.int32) + j
                        for kk in range(DIM // 16):
                            piece = plsc.load_gather(
                                blk_v.at[r], [iota + kk * 16, j16])
                            pool_v[r, b, pl.ds(kk * 16, 16)] = piece
                        pltpu.async_copy(
                            pool_v.at[r, b, pl.ds(0, DIM)],
                            out_hbm.at[pl.ds(p * DIM, DIM)], rsems[r])
                        return rows + 1

                    nrows = lax.fori_loop(seg_lo, seg_hi, row, 0)
                    wraps = jnp.where(nrows > 0, (nrows - 1) // POOL, 0)
                    new_outst.append(nrows - wraps * POOL)
                    fetch(r, (i + 1) * RING + r)
                return tuple(new_outst)

            m3 = (m_w + RING - 1) // RING
            outst = lax.fori_loop(0, m3, triad, (0,) * RING)
            for r in range(RING):
                pltpu.make_async_copy(
                    tbl_hbm.at[:, pl.ds(0, 128)], blk_v.at[r], bsems[r]).wait()

                def drain1(_j, _c):
                    pltpu.make_async_copy(
                        out_hbm.at[pl.ds(0, DIM)],
                        pool_v.at[r, 0, pl.ds(0, DIM)], rsems[r]).wait()
                    return 0

                lax.fori_loop(0, outst[r], drain1, 0)

    return k(vertex_ids, tbl_t, tail_t)


def _sc_label_gather(labels, label_emb_w):
    CH = 128
    NCH = (B // NW) // CH
    lab3 = labels.reshape(NW, NCH, CH)
    mesh = plsc.VectorSubcoreMesh(
        core_axis_name="c", subcore_axis_name="s", num_cores=NC, num_subcores=NS)

    @functools.partial(
        pl.kernel,
        mesh=mesh,
        compiler_params=pltpu.CompilerParams(use_tc_tiling_on_sc=False),
        out_type=jax.ShapeDtypeStruct((B, DIM), jnp.float32),
        scratch_types=[
            pltpu.VMEM((NCH, CH), jnp.int32),
            pltpu.VMEM((NCH * CH, DIM), jnp.float32),
            pltpu.SemaphoreType.DMA,
        ],
    )
    def k(lab_hbm, tbl_hbm, out_hbm, lidx, rows_v, sem):
        wid = lax.axis_index("s") * NC + lax.axis_index("c")
        base = wid * (NCH * CH)
        pltpu.sync_copy(lab_hbm.at[wid], lidx)
        copies = []
        for j in range(NCH):
            dst = rows_v.at[pl.ds(j * CH, CH)]
            copies.append(pltpu.async_copy(tbl_hbm.at[lidx.at[j]], dst, sem))
        for c in copies:
            c.wait()
        pltpu.sync_copy(rows_v, out_hbm.at[pl.ds(base, NCH * CH)])

    return k(lab3, label_emb_w)


def _tc_fuse(id_vec, label_vec, degree, deg_w, deg_b, ln_w, ln_b):
    BLK = 2048
    deg2 = degree.reshape(B, 1)
    wrow = deg_w.reshape(1, DIM)
    brow = deg_b.reshape(1, DIM)
    lwrow = ln_w.reshape(1, DIM)
    lbrow = ln_b.reshape(1, DIM)

    def body(x_ref, y_ref, d_ref, w_ref, b_ref, lw_ref, lb_ref, o_ref):
        d = jnp.clip(d_ref[...], 0.0, 1000000.0)
        d = jnp.log1p(d)
        x = x_ref[...] + y_ref[...] + d * w_ref[...] + b_ref[...]
        mu = jnp.mean(x, axis=-1, keepdims=True)
        xc = x - mu
        var = jnp.mean(xc * xc, axis=-1, keepdims=True)
        xh = xc * lax.rsqrt(var + 1e-5) * lw_ref[...] + lb_ref[...]
        o_ref[...] = 0.5 * xh * (1.0 + lax.erf(xh * 0.7071067811865476))

    small = pl.BlockSpec((1, DIM), lambda i: (0, 0))
    return pl.pallas_call(
        body,
        grid=(B // BLK,),
        in_specs=[
            pl.BlockSpec((BLK, DIM), lambda i: (i, 0)),
            pl.BlockSpec((BLK, DIM), lambda i: (i, 0)),
            pl.BlockSpec((BLK, 1), lambda i: (i, 0)),
            small, small, small, small,
        ],
        out_specs=pl.BlockSpec((BLK, DIM), lambda i: (i, 0)),
        out_shape=jax.ShapeDtypeStruct((B, DIM), jnp.float32),
    )(id_vec, label_vec, deg2, wrow, brow, lwrow, lbrow)


def kernel(vertex_ids, labels, degree, id_emb_w, label_emb_w, deg_w, deg_b, ln_w, ln_b):
    id_vec = _sc_id_gather(vertex_ids, id_emb_w).reshape(B, DIM)
    label_vec = _sc_label_gather(labels, label_emb_w)
    return _tc_fuse(id_vec, label_vec, degree, deg_w, deg_b, ln_w, ln_b)


# speedup vs baseline: 1.2099x; 1.2099x over previous
import functools

import jax
import jax.numpy as jnp
from jax import lax
from jax.experimental import pallas as pl
from jax.experimental.pallas import tpu as pltpu
from jax.experimental.pallas import tpu_sc as plsc

B = 16384
DIM = 64
NV = 1000000
NC, NS = 2, 16
NW = NC * NS
NGT = (NV + 127) // 128
GPW = 245
RING = 4
POOL = 16
ROW_BYTES = DIM * 4


def _sc_id_gather(vertex_ids, id_emb_w):
    tbl_t = id_emb_w.T
    tail_t = id_emb_w[NV - 128:].T
    mesh = plsc.VectorSubcoreMesh(
        core_axis_name="c", subcore_axis_name="s", num_cores=NC, num_subcores=NS)

    @functools.partial(
        pl.kernel,
        mesh=mesh,
        compiler_params=pltpu.CompilerParams(
            needs_layout_passes=False, disable_bounds_checks=True),
        out_type=jax.ShapeDtypeStruct((B * DIM,), jnp.float32),
        scratch_types=[
            pltpu.VMEM((B,), jnp.int32),
            pltpu.VMEM((B + 16,), jnp.int32),
            pltpu.VMEM((B + 16,), jnp.int32),
            pltpu.VMEM((256,), jnp.int32),
            pltpu.VMEM((256,), jnp.int32),
            pltpu.VMEM((256,), jnp.int32),
            pltpu.VMEM((272,), jnp.int32),
            pltpu.VMEM((RING, DIM, 128), jnp.float32),
            pltpu.VMEM((RING, POOL, DIM), jnp.float32),
            pltpu.SemaphoreType.DMA,
            pltpu.SemaphoreType.DMA,
            pltpu.SemaphoreType.DMA,
            pltpu.SemaphoreType.DMA,
            pltpu.SemaphoreType.DMA,
            pltpu.SemaphoreType.DMA,
            pltpu.SemaphoreType.DMA,
            pltpu.SemaphoreType.DMA,
        ],
    )
    def k(ids_hbm, tbl_hbm, tail_hbm, out_hbm, ids_v, spack_v, ssort_v,
          cnt_v, end_v, wr_v, nzg_v, blk_v, pool_v,
          bsem0, bsem1, bsem2, bsem3, rsem0, rsem1, rsem2, rsem3):
        bsems = (bsem0, bsem1, bsem2, bsem3)
        rsems = (rsem0, rsem1, rsem2, rsem3)
        wid = lax.axis_index("s") * NC + lax.axis_index("c")
        g_lo = wid * GPW
        g_hi = jnp.minimum(g_lo + GPW, NGT)
        iota = lax.iota(jnp.int32, 16)
        ones = jnp.ones((16,), jnp.int32)
        zeros = jnp.zeros((16,), jnp.int32)

        pltpu.sync_copy(ids_hbm, ids_v)
        for kk in range(16):
            cnt_v[pl.ds(kk * 16, 16)] = zeros

        def filt(i, off):
            v = ids_v[pl.ds(i * 16, 16)]
            grp = lax.shift_right_logical(v, 7)
            m = (grp >= g_lo) & (grp < g_hi)
            c = plsc.all_reduce_population_count(m)
            lane = (v & 127) + jnp.where(grp == NGT - 1, 64, 0)
            gl = jnp.where(m, grp - g_lo, 0)
            packed = (gl << 21) | ((i * 16 + iota) << 7) | lane
            plsc.store_compressed(spack_v.at[pl.ds(off, 16)], packed, mask=m)
            plsc.addupdate_scatter(cnt_v, [gl], ones, mask=m)
            return off + c[0]

        n = lax.fori_loop(0, B // 16, filt, 0)

        carry = 0
        moff = 0
        for kk in range(16):
            v = cnt_v[pl.ds(kk * 16, 16)]
            cs = plsc.cumsum(v) + carry
            end_v[pl.ds(kk * 16, 16)] = cs
            wr_v[pl.ds(kk * 16, 16)] = cs - v
            m = v > 0
            c = plsc.all_reduce_population_count(m)
            plsc.store_compressed(nzg_v.at[pl.ds(moff, 16)], kk * 16 + iota, mask=m)
            moff = moff + c[0]
            carry = cs[15]
        m_w = moff

        def srt(i, _):
            e = spack_v[pl.ds(i * 16, 16)]
            valid = iota < (n - i * 16)
            gl = jnp.where(valid, lax.shift_right_logical(e, 21), 0)
            base = plsc.load_gather(wr_v, [gl])
            rank, _last = plsc.scan_count(gl, mask=valid)
            dst = base + rank - 1
            plsc.store_scatter(ssort_v, [dst], e, mask=valid)
            plsc.addupdate_scatter(wr_v, [gl], ones, mask=valid)
            return 0

        lax.fori_loop(0, (n + 15) >> 4, srt, 0)

        def fetch(slot, list_idx):
            gl = nzg_v[pl.ds(jnp.minimum(list_idx, m_w - 1), 16)][0]
            gabs = g_lo + gl

            @pl.when(gabs < NGT - 1)
            def _():
                pltpu.async_copy(
                    tbl_hbm.at[:, pl.ds(gabs * 128, 128)], blk_v.at[slot],
                    bsems[slot])

            @pl.when(gabs == NGT - 1)
            def _():
                pltpu.async_copy(tail_hbm, blk_v.at[slot], bsems[slot])

        @pl.when(m_w > 0)
        def _():
            for r in range(RING):
                fetch(r, r)

            def triad(i, outst):
                new_outst = []
                for r in range(RING):
                    o_r = outst[r]

                    def drain1(_j, _c):
                        pltpu.make_async_copy(
                            out_hbm.at[pl.ds(0, DIM)],
                            pool_v.at[r, 0, pl.ds(0, DIM)], rsems[r]).wait()
                        return 0

                    lax.fori_loop(0, o_r, drain1, 0)
                    pltpu.make_async_copy(
                        tbl_hbm.at[:, pl.ds(0, 128)], blk_v.at[r],
                        bsems[r]).wait()

                    idx = jnp.minimum(i * RING + r, m_w - 1)
                    gl = nzg_v[pl.ds(idx, 16)][0]
                    e_prev = end_v[pl.ds(jnp.maximum(gl - 1, 0), 16)][0]
                    seg_lo = jnp.where(gl == 0, 0, e_prev)
                    seg_hi = end_v[pl.ds(gl, 16)][0]

                    def row(s, rows):
                        b = lax.rem(rows, POOL)

                        @pl.when((b == 0) & (rows > 0))
                        def _():
                            lax.fori_loop(0, POOL, drain1, 0)

                        e = ssort_v[pl.ds(s, 16)][0]
                        j = e & 127
                        p = lax.shift_right_logical(e, 7) & (B - 1)
                        j16 = jnp.full((16,), 0, jnp.int32) + j
                        for kk in range(DIM // 16):
                            piece = plsc.load_gather(
                                blk_v.at[r], [iota + kk * 16, j16])
                            pool_v[r, b, pl.ds(kk * 16, 16)] = piece
                        pltpu.async_copy(
                            pool_v.at[r, b, pl.ds(0, DIM)],
                            out_hbm.at[pl.ds(p * DIM, DIM)], rsems[r])
                        return rows + 1

                    nrows = lax.fori_loop(seg_lo, seg_hi, row, 0)
                    wraps = jnp.where(nrows > 0, (nrows - 1) // POOL, 0)
                    new_outst.append(nrows - wraps * POOL)
                    fetch(r, (i + 1) * RING + r)
                return tuple(new_outst)

            m3 = (m_w + RING - 1) // RING
            outst = lax.fori_loop(0, m3, triad, (0,) * RING)
            for r in range(RING):
                pltpu.make_async_copy(
                    tbl_hbm.at[:, pl.ds(0, 128)], blk_v.at[r], bsems[r]).wait()

                def drain1(_j, _c):
                    pltpu.make_async_copy(
                        out_hbm.at[pl.ds(0, DIM)],
                        pool_v.at[r, 0, pl.ds(0, DIM)], rsems[r]).wait()
                    return 0

                lax.fori_loop(0, outst[r], drain1, 0)

    return k(vertex_ids, tbl_t, tail_t)


def _sc_label_gather(labels, label_emb_w):
    CH = 128
    NCH = (B // NW) // CH
    lab3 = labels.reshape(NW, NCH, CH)
    mesh = plsc.VectorSubcoreMesh(
        core_axis_name="c", subcore_axis_name="s", num_cores=NC, num_subcores=NS)

    @functools.partial(
        pl.kernel,
        mesh=mesh,
        compiler_params=pltpu.CompilerParams(use_tc_tiling_on_sc=False),
        out_type=jax.ShapeDtypeStruct((B, DIM), jnp.float32),
        scratch_types=[
            pltpu.VMEM((NCH, CH), jnp.int32),
            pltpu.VMEM((NCH * CH, DIM), jnp.float32),
            pltpu.SemaphoreType.DMA,
        ],
    )
    def k(lab_hbm, tbl_hbm, out_hbm, lidx, rows_v, sem):
        wid = lax.axis_index("s") * NC + lax.axis_index("c")
        base = wid * (NCH * CH)
        pltpu.sync_copy(lab_hbm.at[wid], lidx)
        copies = []
        for j in range(NCH):
            dst = rows_v.at[pl.ds(j * CH, CH)]
            copies.append(pltpu.async_copy(tbl_hbm.at[lidx.at[j]], dst, sem))
        for c in copies:
            c.wait()
        pltpu.sync_copy(rows_v, out_hbm.at[pl.ds(base, NCH * CH)])

    return k(lab3, label_emb_w)


def _tc_fuse(id_vec, label_vec, degree, deg_w, deg_b, ln_w, ln_b):
    BLK = 2048
    deg2 = degree.reshape(B, 1)
    wrow = deg_w.reshape(1, DIM)
    brow = deg_b.reshape(1, DIM)
    lwrow = ln_w.reshape(1, DIM)
    lbrow = ln_b.reshape(1, DIM)

    def body(x_ref, y_ref, d_ref, w_ref, b_ref, lw_ref, lb_ref, o_ref):
        d = jnp.clip(d_ref[...], 0.0, 1000000.0)
        d = jnp.log1p(d)
        x = x_ref[...] + y_ref[...] + d * w_ref[...] + b_ref[...]
        mu = jnp.mean(x, axis=-1, keepdims=True)
        xc = x - mu
        var = jnp.mean(xc * xc, axis=-1, keepdims=True)
        xh = xc * lax.rsqrt(var + 1e-5) * lw_ref[...] + lb_ref[...]
        o_ref[...] = 0.5 * xh * (1.0 + lax.erf(xh * 0.7071067811865476))

    small = pl.BlockSpec((1, DIM), lambda i: (0, 0))
    return pl.pallas_call(
        body,
        grid=(B // BLK,),
        in_specs=[
            pl.BlockSpec((BLK, DIM), lambda i: (i, 0)),
            pl.BlockSpec((BLK, DIM), lambda i: (i, 0)),
            pl.BlockSpec((BLK, 1), lambda i: (i, 0)),
            small, small, small, small,
        ],
        out_specs=pl.BlockSpec((BLK, DIM), lambda i: (i, 0)),
        out_shape=jax.ShapeDtypeStruct((B, DIM), jnp.float32),
    )(id_vec, label_vec, deg2, wrow, brow, lwrow, lbrow)


def kernel(vertex_ids, labels, degree, id_emb_w, label_emb_w, deg_w, deg_b, ln_w, ln_b):
    id_vec = _sc_id_gather(vertex_ids, id_emb_w).reshape(B, DIM)
    label_vec = _sc_label_gather(labels, label_emb_w)
    return _tc_fuse(id_vec, label_vec, degree, deg_w, deg_b, ln_w, ln_b)


# speedup vs baseline: 1.2649x; 1.0454x over previous
import functools

import jax
import jax.numpy as jnp
from jax import lax
from jax.experimental import pallas as pl
from jax.experimental.pallas import tpu as pltpu
from jax.experimental.pallas import tpu_sc as plsc

B = 16384
DIM = 64
NV = 1000000
NC, NS = 2, 16
NW = NC * NS
NGT = (NV + 127) // 128
GPW = 245
RING = 5
POOL = 16
ROW_BYTES = DIM * 4


def _sc_id_gather(vertex_ids, id_emb_w):
    tbl_t = id_emb_w.T
    tail_t = id_emb_w[NV - 128:].T
    mesh = plsc.VectorSubcoreMesh(
        core_axis_name="c", subcore_axis_name="s", num_cores=NC, num_subcores=NS)

    @functools.partial(
        pl.kernel,
        mesh=mesh,
        compiler_params=pltpu.CompilerParams(
            needs_layout_passes=False, disable_bounds_checks=True),
        out_type=jax.ShapeDtypeStruct((B * DIM,), jnp.float32),
        scratch_types=[
            pltpu.VMEM((B,), jnp.int32),
            pltpu.VMEM((B + 16,), jnp.int32),
            pltpu.VMEM((B + 16,), jnp.int32),
            pltpu.VMEM((256,), jnp.int32),
            pltpu.VMEM((256,), jnp.int32),
            pltpu.VMEM((256,), jnp.int32),
            pltpu.VMEM((272,), jnp.int32),
            pltpu.VMEM((RING, DIM, 128), jnp.float32),
            pltpu.VMEM((RING, POOL, DIM), jnp.float32),
            pltpu.SemaphoreType.DMA,
            pltpu.SemaphoreType.DMA,
            pltpu.SemaphoreType.DMA,
            pltpu.SemaphoreType.DMA,
            pltpu.SemaphoreType.DMA,
            pltpu.SemaphoreType.DMA,
            pltpu.SemaphoreType.DMA,
            pltpu.SemaphoreType.DMA,
            pltpu.SemaphoreType.DMA,
            pltpu.SemaphoreType.DMA,
        ],
    )
    def k(ids_hbm, tbl_hbm, tail_hbm, out_hbm, ids_v, spack_v, ssort_v,
          cnt_v, end_v, wr_v, nzg_v, blk_v, pool_v,
          bsem0, bsem1, bsem2, bsem3, bsem4,
          rsem0, rsem1, rsem2, rsem3, rsem4):
        bsems = (bsem0, bsem1, bsem2, bsem3, bsem4)
        rsems = (rsem0, rsem1, rsem2, rsem3, rsem4)
        wid = lax.axis_index("s") * NC + lax.axis_index("c")
        g_lo = wid * GPW
        g_hi = jnp.minimum(g_lo + GPW, NGT)
        iota = lax.iota(jnp.int32, 16)
        ones = jnp.ones((16,), jnp.int32)
        zeros = jnp.zeros((16,), jnp.int32)

        pltpu.sync_copy(ids_hbm, ids_v)
        for kk in range(16):
            cnt_v[pl.ds(kk * 16, 16)] = zeros

        def filt(i, off):
            v = ids_v[pl.ds(i * 16, 16)]
            grp = lax.shift_right_logical(v, 7)
            m = (grp >= g_lo) & (grp < g_hi)
            c = plsc.all_reduce_population_count(m)
            lane = (v & 127) + jnp.where(grp == NGT - 1, 64, 0)
            gl = jnp.where(m, grp - g_lo, 0)
            packed = (gl << 21) | ((i * 16 + iota) << 7) | lane
            plsc.store_compressed(spack_v.at[pl.ds(off, 16)], packed, mask=m)
            plsc.addupdate_scatter(cnt_v, [gl], ones, mask=m)
            return off + c[0]

        n = lax.fori_loop(0, B // 16, filt, 0)

        carry = 0
        moff = 0
        for kk in range(16):
            v = cnt_v[pl.ds(kk * 16, 16)]
            cs = plsc.cumsum(v) + carry
            end_v[pl.ds(kk * 16, 16)] = cs
            wr_v[pl.ds(kk * 16, 16)] = cs - v
            m = v > 0
            c = plsc.all_reduce_population_count(m)
            plsc.store_compressed(nzg_v.at[pl.ds(moff, 16)], kk * 16 + iota, mask=m)
            moff = moff + c[0]
            carry = cs[15]
        m_w = moff

        def srt(i, _):
            e = spack_v[pl.ds(i * 16, 16)]
            valid = iota < (n - i * 16)
            gl = jnp.where(valid, lax.shift_right_logical(e, 21), 0)
            base = plsc.load_gather(wr_v, [gl])
            rank, _last = plsc.scan_count(gl, mask=valid)
            dst = base + rank - 1
            plsc.store_scatter(ssort_v, [dst], e, mask=valid)
            plsc.addupdate_scatter(wr_v, [gl], ones, mask=valid)
            return 0

        lax.fori_loop(0, (n + 15) >> 4, srt, 0)

        def fetch(slot, list_idx):
            gl = nzg_v[pl.ds(jnp.minimum(list_idx, m_w - 1), 16)][0]
            gabs = g_lo + gl

            @pl.when(gabs < NGT - 1)
            def _():
                pltpu.async_copy(
                    tbl_hbm.at[:, pl.ds(gabs * 128, 128)], blk_v.at[slot],
                    bsems[slot])

            @pl.when(gabs == NGT - 1)
            def _():
                pltpu.async_copy(tail_hbm, blk_v.at[slot], bsems[slot])

        @pl.when(m_w > 0)
        def _():
            for r in range(RING):
                fetch(r, r)

            def triad(i, outst):
                new_outst = []
                for r in range(RING):
                    o_r = outst[r]

                    def drain1(_j, _c):
                        pltpu.make_async_copy(
                            out_hbm.at[pl.ds(0, DIM)],
                            pool_v.at[r, 0, pl.ds(0, DIM)], rsems[r]).wait()
                        return 0

                    lax.fori_loop(0, o_r, drain1, 0)
                    pltpu.make_async_copy(
                        tbl_hbm.at[:, pl.ds(0, 128)], blk_v.at[r],
                        bsems[r]).wait()

                    idx = jnp.minimum(i * RING + r, m_w - 1)
                    gl = nzg_v[pl.ds(idx, 16)][0]
                    e_prev = end_v[pl.ds(jnp.maximum(gl - 1, 0), 16)][0]
                    seg_lo = jnp.where(gl == 0, 0, e_prev)
                    seg_hi = end_v[pl.ds(gl, 16)][0]

                    def row(s, rows):
                        b = lax.rem(rows, POOL)

                        @pl.when((b == 0) & (rows > 0))
                        def _():
                            lax.fori_loop(0, POOL, drain1, 0)

                        e = ssort_v[pl.ds(s, 16)][0]
                        j = e & 127
                        p = lax.shift_right_logical(e, 7) & (B - 1)
                        j16 = jnp.full((16,), 0, jnp.int32) + j
                        for kk in range(DIM // 16):
                            piece = plsc.load_gather(
                                blk_v.at[r], [iota + kk * 16, j16])
                            pool_v[r, b, pl.ds(kk * 16, 16)] = piece
                        pltpu.async_copy(
                            pool_v.at[r, b, pl.ds(0, DIM)],
                            out_hbm.at[pl.ds(p * DIM, DIM)], rsems[r])
                        return rows + 1

                    nrows = lax.fori_loop(seg_lo, seg_hi, row, 0)
                    wraps = jnp.where(nrows > 0, (nrows - 1) // POOL, 0)
                    new_outst.append(nrows - wraps * POOL)
                    fetch(r, (i + 1) * RING + r)
                return tuple(new_outst)

            m3 = (m_w + RING - 1) // RING
            outst = lax.fori_loop(0, m3, triad, (0,) * RING)
            for r in range(RING):
                pltpu.make_async_copy(
                    tbl_hbm.at[:, pl.ds(0, 128)], blk_v.at[r], bsems[r]).wait()

                def drain1(_j, _c):
                    pltpu.make_async_copy(
                        out_hbm.at[pl.ds(0, DIM)],
                        pool_v.at[r, 0, pl.ds(0, DIM)], rsems[r]).wait()
                    return 0

                lax.fori_loop(0, outst[r], drain1, 0)

    return k(vertex_ids, tbl_t, tail_t)


def _sc_label_gather(labels, label_emb_w):
    CH = 128
    NCH = (B // NW) // CH
    lab3 = labels.reshape(NW, NCH, CH)
    mesh = plsc.VectorSubcoreMesh(
        core_axis_name="c", subcore_axis_name="s", num_cores=NC, num_subcores=NS)

    @functools.partial(
        pl.kernel,
        mesh=mesh,
        compiler_params=pltpu.CompilerParams(use_tc_tiling_on_sc=False),
        out_type=jax.ShapeDtypeStruct((B, DIM), jnp.float32),
        scratch_types=[
            pltpu.VMEM((NCH, CH), jnp.int32),
            pltpu.VMEM((NCH * CH, DIM), jnp.float32),
            pltpu.SemaphoreType.DMA,
        ],
    )
    def k(lab_hbm, tbl_hbm, out_hbm, lidx, rows_v, sem):
        wid = lax.axis_index("s") * NC + lax.axis_index("c")
        base = wid * (NCH * CH)
        pltpu.sync_copy(lab_hbm.at[wid], lidx)
        copies = []
        for j in range(NCH):
            dst = rows_v.at[pl.ds(j * CH, CH)]
            copies.append(pltpu.async_copy(tbl_hbm.at[lidx.at[j]], dst, sem))
        for c in copies:
            c.wait()
        pltpu.sync_copy(rows_v, out_hbm.at[pl.ds(base, NCH * CH)])

    return k(lab3, label_emb_w)


def _tc_fuse(id_vec, label_vec, degree, deg_w, deg_b, ln_w, ln_b):
    BLK = 2048
    deg2 = degree.reshape(B, 1)
    wrow = deg_w.reshape(1, DIM)
    brow = deg_b.reshape(1, DIM)
    lwrow = ln_w.reshape(1, DIM)
    lbrow = ln_b.reshape(1, DIM)

    def body(x_ref, y_ref, d_ref, w_ref, b_ref, lw_ref, lb_ref, o_ref):
        d = jnp.clip(d_ref[...], 0.0, 1000000.0)
        d = jnp.log1p(d)
        x = x_ref[...] + y_ref[...] + d * w_ref[...] + b_ref[...]
        mu = jnp.mean(x, axis=-1, keepdims=True)
        xc = x - mu
        var = jnp.mean(xc * xc, axis=-1, keepdims=True)
        xh = xc * lax.rsqrt(var + 1e-5) * lw_ref[...] + lb_ref[...]
        o_ref[...] = 0.5 * xh * (1.0 + lax.erf(xh * 0.7071067811865476))

    small = pl.BlockSpec((1, DIM), lambda i: (0, 0))
    return pl.pallas_call(
        body,
        grid=(B // BLK,),
        in_specs=[
            pl.BlockSpec((BLK, DIM), lambda i: (i, 0)),
            pl.BlockSpec((BLK, DIM), lambda i: (i, 0)),
            pl.BlockSpec((BLK, 1), lambda i: (i, 0)),
            small, small, small, small,
        ],
        out_specs=pl.BlockSpec((BLK, DIM), lambda i: (i, 0)),
        out_shape=jax.ShapeDtypeStruct((B, DIM), jnp.float32),
    )(id_vec, label_vec, deg2, wrow, brow, lwrow, lbrow)


def kernel(vertex_ids, labels, degree, id_emb_w, label_emb_w, deg_w, deg_b, ln_w, ln_b):
    id_vec = _sc_id_gather(vertex_ids, id_emb_w).reshape(B, DIM)
    label_vec = _sc_label_gather(labels, label_emb_w)
    return _tc_fuse(id_vec, label_vec, degree, deg_w, deg_b, ln_w, ln_b)
